# Initial kernel scaffold; baseline (speedup 1.0000x reference)
import jax
import jax.numpy as jnp
from jax.experimental import pallas as pl


def kernel(input_ids, attention_mask, table):
    raise NotImplementedError("write your pallas kernel here")


# speedup vs baseline: 1.5518x; 1.5518x over previous
import functools

import jax
import jax.numpy as jnp
from jax import lax
from jax.experimental import pallas as pl
from jax.experimental.pallas import tpu as pltpu
from jax.experimental.pallas import tpu_sc as plsc

VOCAB = 151936
HIDDEN = 1536
BATCH = 4
SEQ = 2048

_info = plsc.get_sparse_core_info()
_NC, _NS = _info.num_cores, _info.num_subcores
_NW = _NC * _NS
_NTOT = BATCH * SEQ
_BPW = _NTOT // _NW
_CHUNK = 64
_NCHUNK = _BPW // _CHUNK


@functools.partial(
    pl.kernel,
    mesh=plsc.VectorSubcoreMesh(core_axis_name="c", subcore_axis_name="s"),
    out_type=jax.ShapeDtypeStruct((_NTOT, HIDDEN), jnp.float32),
    scratch_types=[
        pltpu.VMEM((_CHUNK,), jnp.int32),
        pltpu.VMEM((_CHUNK, HIDDEN), jnp.float32),
        pltpu.SemaphoreType.DMA,
    ],
)
def _gather_rows(table_hbm, ids_hbm, out_hbm, idx_v, rows_v, sem):
    wid = lax.axis_index("s") * _NC + lax.axis_index("c")
    base = wid * _BPW
    for j in range(_NCHUNK):
        off = base + j * _CHUNK
        pltpu.sync_copy(ids_hbm.at[pl.ds(off, _CHUNK)], idx_v)
        pltpu.async_copy(table_hbm.at[idx_v], rows_v, sem).wait()
        pltpu.sync_copy(rows_v, out_hbm.at[pl.ds(off, _CHUNK)])


def kernel(input_ids, attention_mask, table):
    ids_flat = input_ids.reshape(_NTOT)
    out_flat = _gather_rows(table, ids_flat)
    hidden_states = out_flat.reshape(BATCH, SEQ, HIDDEN)
    seq_lengths = attention_mask.sum(axis=1) - 1
    last_hidden = hidden_states[jnp.arange(BATCH), seq_lengths]
    return (hidden_states, last_hidden)
